# baseline (speedup 1.0000x reference)
import numpy as np
import jax
import jax.numpy as jnp
from jax.experimental import pallas as pl

B = 16
L = 1024
PL = 16
PS = 4
LW = 4
H = 128
NC = 8
NL = 4
P = (L - PL) // PS + 1
NPS = P * PL
NPAD = 4096
BT = 256
NT = NPAD // BT
NS = 8
NR = NS * NPAD


def _build_agg_mats():
    bd = np.zeros((BT, BT), dtype=np.float64)
    for r in range(BT):
        j = r % PL
        p0 = r - j
        for i in range(PL):
            if 0 < abs(i - j) <= LW:
                bd[r, p0 + i] = 1.0
        if j == 0 and r > 0:
            bd[r, r - 1] = 1.0
    deg = np.minimum(np.arange(BT) % PL, LW) \
        + np.minimum(PL - 1 - np.arange(BT) % PL, LW) \
        + ((np.arange(BT) % PL) == 0)
    bds = bd / deg[:, None]
    bd0 = bds.copy()
    bd0[0, :] = bd[0, :] / (deg[0] - 1)
    bde = np.zeros((BT, 2 * BT), dtype=np.float64)
    bde[:, BT:] = bds
    bde[0, BT - 1] = 1.0 / deg[0]
    return bd0.astype(np.float32), bde.astype(np.float32)

_BD0_NP, _BDE_NP = _build_agg_mats()

YW = 128
YOFF = 32


def _build_window_mats():
    sw = np.zeros((BT, YW), dtype=np.float64)
    bds = np.zeros((BT, YW), dtype=np.float64)
    bds0 = np.zeros((BT, YW), dtype=np.float64)

    def ycol(r):
        return 4 * (r // PL) + (r % PL) + YOFF

    deg = np.minimum(np.arange(BT) % PL, LW) \
        + np.minimum(PL - 1 - np.arange(BT) % PL, LW) \
        + ((np.arange(BT) % PL) == 0)
    for r in range(BT):
        sw[r, ycol(r)] = 1.0
        j = r % PL
        p0 = r - j
        for i in range(PL):
            if 0 < abs(i - j) <= LW:
                bds[r, ycol(p0 + i)] += 1.0 / deg[r]
                bds0[r, ycol(p0 + i)] += 1.0 / (deg[r] if r > 0 else deg[r] - 1)
        if j == 0:
            bds[r, ycol(r - 1)] += 1.0 / deg[r]
            if r > 0:
                bds0[r, ycol(r - 1)] += 1.0 / deg[r]
    return (sw.astype(np.float32), bds.astype(np.float32),
            bds0.astype(np.float32))

_SW_NP, _BDS_NP, _BDS0_NP = _build_window_mats()


def _ln_relu(h, g, b):
    mu = jnp.mean(h, axis=-1, keepdims=True)
    var = jnp.mean((h - mu) ** 2, axis=-1, keepdims=True)
    y = (h - mu) * jax.lax.rsqrt(var + 1e-5) * g + b
    return jnp.maximum(y, 0.0)


def _banded_mean_bf16(ub, bde_ref, bd0_ref):
    parts = []
    for t in range(NS * NT):
        if t % NT == 0:
            parts.append(jnp.dot(bd0_ref[...], ub[t * BT:(t + 1) * BT],
                                 preferred_element_type=jnp.float32))
        else:
            parts.append(jnp.dot(bde_ref[...], ub[(t - 1) * BT:(t + 1) * BT],
                                 preferred_element_type=jnp.float32))
    return jnp.concatenate(parts, axis=0)


def _gnn_kernel(iqt_ref, w0_ref, sw_ref, bds_ref, bds0_ref,
                bde_ref, bd0_ref, wl_ref, wr_ref,
                b_ref, g_ref, bb_ref, out_ref):
    iqt = iqt_ref[0]
    yuv = jnp.dot(iqt.astype(jnp.bfloat16), w0_ref[...],
                  preferred_element_type=jnp.float32)
    mean_parts = []
    v_parts = []
    zpad = jnp.zeros((YOFF, 2 * H), jnp.float32)
    for s in range(NS):
        yb = jnp.concatenate(
            [zpad, yuv[s * L:(s + 1) * L], zpad], axis=0
        ).astype(jnp.bfloat16)
        for t in range(NT):
            win = yb[64 * t:64 * t + YW]
            bdsm = bds0_ref[...] if t == 0 else bds_ref[...]
            mean_parts.append(jnp.dot(bdsm, win[:, :H],
                                      preferred_element_type=jnp.float32))
            v_parts.append(jnp.dot(sw_ref[...], win[:, H:],
                                   preferred_element_type=jnp.float32))
    h = jnp.concatenate(mean_parts, axis=0) \
        + jnp.concatenate(v_parts, axis=0) + b_ref[0:1, :]
    x = _ln_relu(h, g_ref[0:1, :], bb_ref[0:1, :])
    for i in range(1, NL):
        xb = x.astype(jnp.bfloat16)
        m = _banded_mean_bf16(xb, bde_ref, bd0_ref).astype(jnp.bfloat16)
        h = jnp.dot(m, wl_ref[i - 1], preferred_element_type=jnp.float32) \
            + jnp.dot(xb, wr_ref[i - 1], preferred_element_type=jnp.float32) \
            + b_ref[i:i + 1, :]
        x = x + _ln_relu(h, g_ref[i:i + 1, :], bb_ref[i:i + 1, :])
    for s in range(NS):
        out_ref[0, s] = jnp.mean(x[s * NPAD:s * NPAD + NPS], axis=0)


def _cls_kernel(sig_ref, w1_ref, b1_ref, g_ref, be_ref, w2_ref, b2_ref,
                out_ref):
    h = jnp.dot(sig_ref[...], w1_ref[...],
                preferred_element_type=jnp.float32) + b1_ref[...]
    h = _ln_relu(h, g_ref[...], be_ref[...])
    out_ref[...] = jnp.dot(h, w2_ref[...],
                           preferred_element_type=jnp.float32) + b2_ref[...]


def kernel(iq_signal, params):
    iqt = jnp.transpose(iq_signal, (0, 2, 1))
    iqt = iqt.reshape(B // NS, NS * L, 2)

    bde = jnp.asarray(_BDE_NP).astype(jnp.bfloat16)
    bd0 = jnp.asarray(_BD0_NP).astype(jnp.bfloat16)
    sw = jnp.asarray(_SW_NP).astype(jnp.bfloat16)
    bds = jnp.asarray(_BDS_NP).astype(jnp.bfloat16)
    bds0 = jnp.asarray(_BDS0_NP).astype(jnp.bfloat16)
    w0 = jnp.concatenate([params['sage_Wl_0'], params['sage_Wr_0']],
                         axis=1).astype(jnp.bfloat16)
    wl = jnp.stack([params['sage_Wl_%d' % i]
                    for i in range(1, NL)]).astype(jnp.bfloat16)
    wr = jnp.stack([params['sage_Wr_%d' % i]
                    for i in range(1, NL)]).astype(jnp.bfloat16)
    b_all = jnp.stack([params['sage_b_%d' % i] for i in range(NL)])
    g_all = jnp.stack([params['ln_g_%d' % i] for i in range(NL)])
    bb_all = jnp.stack([params['ln_b_%d' % i] for i in range(NL)])

    sig = pl.pallas_call(
        _gnn_kernel,
        grid=(B // NS,),
        in_specs=[
            pl.BlockSpec((1, NS * L, 2), lambda b: (b, 0, 0)),
            pl.BlockSpec((2, 2 * H), lambda b: (0, 0)),
            pl.BlockSpec((BT, YW), lambda b: (0, 0)),
            pl.BlockSpec((BT, YW), lambda b: (0, 0)),
            pl.BlockSpec((BT, YW), lambda b: (0, 0)),
            pl.BlockSpec((BT, 2 * BT), lambda b: (0, 0)),
            pl.BlockSpec((BT, BT), lambda b: (0, 0)),
            pl.BlockSpec((NL - 1, H, H), lambda b: (0, 0, 0)),
            pl.BlockSpec((NL - 1, H, H), lambda b: (0, 0, 0)),
            pl.BlockSpec((NL, H), lambda b: (0, 0)),
            pl.BlockSpec((NL, H), lambda b: (0, 0)),
            pl.BlockSpec((NL, H), lambda b: (0, 0)),
        ],
        out_specs=pl.BlockSpec((1, NS, H), lambda b: (b, 0, 0)),
        out_shape=jax.ShapeDtypeStruct((B // NS, NS, H), jnp.float32),
    )(iqt, w0, sw, bds, bds0, bde, bd0, wl, wr, b_all, g_all, bb_all)
    sig = sig.reshape(B, H)

    logits = pl.pallas_call(
        _cls_kernel,
        in_specs=[
            pl.BlockSpec((B, H), lambda: (0, 0)),
            pl.BlockSpec((H, H), lambda: (0, 0)),
            pl.BlockSpec((1, H), lambda: (0, 0)),
            pl.BlockSpec((1, H), lambda: (0, 0)),
            pl.BlockSpec((1, H), lambda: (0, 0)),
            pl.BlockSpec((H, NC), lambda: (0, 0)),
            pl.BlockSpec((1, NC), lambda: (0, 0)),
        ],
        out_specs=pl.BlockSpec((B, NC), lambda: (0, 0)),
        out_shape=jax.ShapeDtypeStruct((B, NC), jnp.float32),
    )(sig, params['cls_W1'], params['cls_b1'][None, :],
      params['cls_g'][None, :], params['cls_be'][None, :],
      params['cls_W2'], params['cls_b2'][None, :])
    return logits


# speedup vs baseline: 1.3201x; 1.3201x over previous
import numpy as np
import jax
import jax.numpy as jnp
from jax.experimental import pallas as pl

B = 16
L = 1024
PL = 16
PS = 4
LW = 4
H = 128
NC = 8
NL = 4
P = (L - PL) // PS + 1
NPS = P * PL
NPAD = 4096
BT = 256
NT = NPAD // BT
NS = 4
NR = NS * NPAD


def _build_agg_mats():
    bd = np.zeros((BT, BT), dtype=np.float64)
    for r in range(BT):
        j = r % PL
        p0 = r - j
        for i in range(PL):
            if 0 < abs(i - j) <= LW:
                bd[r, p0 + i] = 1.0
        if j == 0 and r > 0:
            bd[r, r - 1] = 1.0
    deg = np.minimum(np.arange(BT) % PL, LW) \
        + np.minimum(PL - 1 - np.arange(BT) % PL, LW) \
        + ((np.arange(BT) % PL) == 0)
    bds = bd / deg[:, None]
    bd0 = bds.copy()
    bd0[0, :] = bd[0, :] / (deg[0] - 1)
    bde = np.zeros((BT, 2 * BT), dtype=np.float64)
    bde[:, BT:] = bds
    bde[0, BT - 1] = 1.0 / deg[0]
    return bd0.astype(np.float32), bde.astype(np.float32)

_BD0_NP, _BDE_NP = _build_agg_mats()

YW = 128
YOFF = 32


def _build_window_mats():
    sw = np.zeros((BT, YW), dtype=np.float64)
    bds = np.zeros((BT, YW), dtype=np.float64)
    bds0 = np.zeros((BT, YW), dtype=np.float64)

    def ycol(r):
        return 4 * (r // PL) + (r % PL) + YOFF

    deg = np.minimum(np.arange(BT) % PL, LW) \
        + np.minimum(PL - 1 - np.arange(BT) % PL, LW) \
        + ((np.arange(BT) % PL) == 0)
    for r in range(BT):
        sw[r, ycol(r)] = 1.0
        j = r % PL
        p0 = r - j
        for i in range(PL):
            if 0 < abs(i - j) <= LW:
                bds[r, ycol(p0 + i)] += 1.0 / deg[r]
                bds0[r, ycol(p0 + i)] += 1.0 / (deg[r] if r > 0 else deg[r] - 1)
        if j == 0:
            bds[r, ycol(r - 1)] += 1.0 / deg[r]
            if r > 0:
                bds0[r, ycol(r - 1)] += 1.0 / deg[r]
    return (sw.astype(np.float32), bds.astype(np.float32),
            bds0.astype(np.float32))

_SW_NP, _BDS_NP, _BDS0_NP = _build_window_mats()


def _ln_relu(h, g, b):
    mu = jnp.mean(h, axis=-1, keepdims=True)
    var = jnp.mean((h - mu) ** 2, axis=-1, keepdims=True)
    y = (h - mu) * jax.lax.rsqrt(var + 1e-5) * g + b
    return jnp.maximum(y, 0.0)


def _banded_mean_bf16(ub, bde_ref, bd0_ref):
    parts = []
    for t in range(NS * NT):
        if t % NT == 0:
            parts.append(jnp.dot(bd0_ref[...], ub[t * BT:(t + 1) * BT],
                                 preferred_element_type=jnp.float32))
        else:
            parts.append(jnp.dot(bde_ref[...], ub[(t - 1) * BT:(t + 1) * BT],
                                 preferred_element_type=jnp.float32))
    return jnp.concatenate(parts, axis=0)


def _gnn_kernel(iq_ref, w0_ref, sw_ref, bds_ref, bds0_ref,
                bde_ref, bd0_ref, wl_ref, wr_ref,
                b_ref, g_ref, bb_ref, out_ref):
    mean_parts = []
    v_parts = []
    zpad = jnp.zeros((YOFF, 2 * H), jnp.float32)
    for s in range(NS):
        yuv = jax.lax.dot_general(
            iq_ref[0, s].astype(jnp.bfloat16), w0_ref[...],
            (((0,), (0,)), ((), ())),
            preferred_element_type=jnp.float32)
        yb = jnp.concatenate([zpad, yuv, zpad], axis=0).astype(jnp.bfloat16)
        for t in range(NT):
            win = yb[64 * t:64 * t + YW]
            bdsm = bds0_ref[...] if t == 0 else bds_ref[...]
            mean_parts.append(jnp.dot(bdsm, win[:, :H],
                                      preferred_element_type=jnp.float32))
            v_parts.append(jnp.dot(sw_ref[...], win[:, H:],
                                   preferred_element_type=jnp.float32))
    h = jnp.concatenate(mean_parts, axis=0) \
        + jnp.concatenate(v_parts, axis=0) + b_ref[0:1, :]
    x = _ln_relu(h, g_ref[0:1, :], bb_ref[0:1, :])
    for i in range(1, NL):
        xb = x.astype(jnp.bfloat16)
        m = _banded_mean_bf16(xb, bde_ref, bd0_ref).astype(jnp.bfloat16)
        h = jnp.dot(m, wl_ref[i - 1], preferred_element_type=jnp.float32) \
            + jnp.dot(xb, wr_ref[i - 1], preferred_element_type=jnp.float32) \
            + b_ref[i:i + 1, :]
        x = x + _ln_relu(h, g_ref[i:i + 1, :], bb_ref[i:i + 1, :])
    for s in range(NS):
        out_ref[0, s] = jnp.mean(x[s * NPAD:s * NPAD + NPS], axis=0)


def _cls_kernel(sig_ref, w1_ref, b1_ref, g_ref, be_ref, w2_ref, b2_ref,
                out_ref):
    h = jnp.dot(sig_ref[...], w1_ref[...],
                preferred_element_type=jnp.float32) + b1_ref[...]
    h = _ln_relu(h, g_ref[...], be_ref[...])
    out_ref[...] = jnp.dot(h, w2_ref[...],
                           preferred_element_type=jnp.float32) + b2_ref[...]


def kernel(iq_signal, params):
    iqg = iq_signal.reshape(B // NS, NS, 2, L)

    bde = jnp.asarray(_BDE_NP).astype(jnp.bfloat16)
    bd0 = jnp.asarray(_BD0_NP).astype(jnp.bfloat16)
    sw = jnp.asarray(_SW_NP).astype(jnp.bfloat16)
    bds = jnp.asarray(_BDS_NP).astype(jnp.bfloat16)
    bds0 = jnp.asarray(_BDS0_NP).astype(jnp.bfloat16)
    w0 = jnp.concatenate([params['sage_Wl_0'], params['sage_Wr_0']],
                         axis=1).astype(jnp.bfloat16)
    wl = jnp.stack([params['sage_Wl_%d' % i]
                    for i in range(1, NL)]).astype(jnp.bfloat16)
    wr = jnp.stack([params['sage_Wr_%d' % i]
                    for i in range(1, NL)]).astype(jnp.bfloat16)
    b_all = jnp.stack([params['sage_b_%d' % i] for i in range(NL)])
    g_all = jnp.stack([params['ln_g_%d' % i] for i in range(NL)])
    bb_all = jnp.stack([params['ln_b_%d' % i] for i in range(NL)])

    sig = pl.pallas_call(
        _gnn_kernel,
        grid=(B // NS,),
        in_specs=[
            pl.BlockSpec((1, NS, 2, L), lambda b: (b, 0, 0, 0)),
            pl.BlockSpec((2, 2 * H), lambda b: (0, 0)),
            pl.BlockSpec((BT, YW), lambda b: (0, 0)),
            pl.BlockSpec((BT, YW), lambda b: (0, 0)),
            pl.BlockSpec((BT, YW), lambda b: (0, 0)),
            pl.BlockSpec((BT, 2 * BT), lambda b: (0, 0)),
            pl.BlockSpec((BT, BT), lambda b: (0, 0)),
            pl.BlockSpec((NL - 1, H, H), lambda b: (0, 0, 0)),
            pl.BlockSpec((NL - 1, H, H), lambda b: (0, 0, 0)),
            pl.BlockSpec((NL, H), lambda b: (0, 0)),
            pl.BlockSpec((NL, H), lambda b: (0, 0)),
            pl.BlockSpec((NL, H), lambda b: (0, 0)),
        ],
        out_specs=pl.BlockSpec((1, NS, H), lambda b: (b, 0, 0)),
        out_shape=jax.ShapeDtypeStruct((B // NS, NS, H), jnp.float32),
    )(iqg, w0, sw, bds, bds0, bde, bd0, wl, wr, b_all, g_all, bb_all)
    sig = sig.reshape(B, H)

    logits = pl.pallas_call(
        _cls_kernel,
        in_specs=[
            pl.BlockSpec((B, H), lambda: (0, 0)),
            pl.BlockSpec((H, H), lambda: (0, 0)),
            pl.BlockSpec((1, H), lambda: (0, 0)),
            pl.BlockSpec((1, H), lambda: (0, 0)),
            pl.BlockSpec((1, H), lambda: (0, 0)),
            pl.BlockSpec((H, NC), lambda: (0, 0)),
            pl.BlockSpec((1, NC), lambda: (0, 0)),
        ],
        out_specs=pl.BlockSpec((B, NC), lambda: (0, 0)),
        out_shape=jax.ShapeDtypeStruct((B, NC), jnp.float32),
    )(sig, params['cls_W1'], params['cls_b1'][None, :],
      params['cls_g'][None, :], params['cls_be'][None, :],
      params['cls_W2'], params['cls_b2'][None, :])
    return logits


# speedup vs baseline: 1.3348x; 1.0111x over previous
import numpy as np
import jax
import jax.numpy as jnp
from jax.experimental import pallas as pl
from jax.experimental.pallas import tpu as pltpu

B = 16
L = 1024
PL = 16
PS = 4
LW = 4
H = 128
NC = 8
NL = 4
P = (L - PL) // PS + 1
NPS = P * PL
NPAD = 4096
BT = 256
NT = NPAD // BT
NS = 4
NR = NS * NPAD


def _build_agg_mats():
    bd = np.zeros((BT, BT), dtype=np.float64)
    for r in range(BT):
        j = r % PL
        p0 = r - j
        for i in range(PL):
            if 0 < abs(i - j) <= LW:
                bd[r, p0 + i] = 1.0
        if j == 0 and r > 0:
            bd[r, r - 1] = 1.0
    deg = np.minimum(np.arange(BT) % PL, LW) \
        + np.minimum(PL - 1 - np.arange(BT) % PL, LW) \
        + ((np.arange(BT) % PL) == 0)
    bds = bd / deg[:, None]
    bd0 = bds.copy()
    bd0[0, :] = bd[0, :] / (deg[0] - 1)
    bde = np.zeros((BT, 2 * BT), dtype=np.float64)
    bde[:, BT:] = bds
    bde[0, BT - 1] = 1.0 / deg[0]
    return bd0.astype(np.float32), bde.astype(np.float32)

_BD0_NP, _BDE_NP = _build_agg_mats()

YW = 128
YOFF = 32


def _build_window_mats():
    sw = np.zeros((BT, YW), dtype=np.float64)
    bds = np.zeros((BT, YW), dtype=np.float64)
    bds0 = np.zeros((BT, YW), dtype=np.float64)

    def ycol(r):
        return 4 * (r // PL) + (r % PL) + YOFF

    deg = np.minimum(np.arange(BT) % PL, LW) \
        + np.minimum(PL - 1 - np.arange(BT) % PL, LW) \
        + ((np.arange(BT) % PL) == 0)
    for r in range(BT):
        sw[r, ycol(r)] = 1.0
        j = r % PL
        p0 = r - j
        for i in range(PL):
            if 0 < abs(i - j) <= LW:
                bds[r, ycol(p0 + i)] += 1.0 / deg[r]
                bds0[r, ycol(p0 + i)] += 1.0 / (deg[r] if r > 0 else deg[r] - 1)
        if j == 0:
            bds[r, ycol(r - 1)] += 1.0 / deg[r]
            if r > 0:
                bds0[r, ycol(r - 1)] += 1.0 / deg[r]
    return (sw.astype(np.float32), bds.astype(np.float32),
            bds0.astype(np.float32))

_SW_NP, _BDS_NP, _BDS0_NP = _build_window_mats()


def _ln_relu(h, g, b):
    mu = jnp.mean(h, axis=-1, keepdims=True)
    var = jnp.mean((h - mu) ** 2, axis=-1, keepdims=True)
    y = (h - mu) * jax.lax.rsqrt(var + 1e-5) * g + b
    return jnp.maximum(y, 0.0)


def _banded_mean_bf16(ub, bde_ref, bd0_ref):
    parts = []
    for t in range(NS * NT):
        if t % NT == 0:
            parts.append(jnp.dot(bd0_ref[...], ub[t * BT:(t + 1) * BT],
                                 preferred_element_type=jnp.float32))
        else:
            parts.append(jnp.dot(bde_ref[...], ub[(t - 1) * BT:(t + 1) * BT],
                                 preferred_element_type=jnp.float32))
    return jnp.concatenate(parts, axis=0)


def _gnn_kernel(iq_ref, w0_ref, sw_ref, bds_ref, bds0_ref,
                bde_ref, bd0_ref, wl_ref, wr_ref,
                b_ref, g_ref, bb_ref,
                w1_ref, b1_ref, cg_ref, cbe_ref, w2_ref, b2_ref,
                out_ref, sig_ref):
    mean_parts = []
    v_parts = []
    zpad = jnp.zeros((YOFF, 2 * H), jnp.float32)
    for s in range(NS):
        yuv = jax.lax.dot_general(
            iq_ref[0, s].astype(jnp.bfloat16), w0_ref[...],
            (((0,), (0,)), ((), ())),
            preferred_element_type=jnp.float32)
        yb = jnp.concatenate([zpad, yuv, zpad], axis=0).astype(jnp.bfloat16)
        for t in range(NT):
            win = yb[64 * t:64 * t + YW]
            bdsm = bds0_ref[...] if t == 0 else bds_ref[...]
            mean_parts.append(jnp.dot(bdsm, win[:, :H],
                                      preferred_element_type=jnp.float32))
            v_parts.append(jnp.dot(sw_ref[...], win[:, H:],
                                   preferred_element_type=jnp.float32))
    h = jnp.concatenate(mean_parts, axis=0) \
        + jnp.concatenate(v_parts, axis=0) + b_ref[0:1, :]
    x = _ln_relu(h, g_ref[0:1, :], bb_ref[0:1, :])
    for i in range(1, NL):
        xb = x.astype(jnp.bfloat16)
        m = _banded_mean_bf16(xb, bde_ref, bd0_ref).astype(jnp.bfloat16)
        h = jnp.dot(m, wl_ref[i - 1], preferred_element_type=jnp.float32) \
            + jnp.dot(xb, wr_ref[i - 1], preferred_element_type=jnp.float32) \
            + b_ref[i:i + 1, :]
        x = x + _ln_relu(h, g_ref[i:i + 1, :], bb_ref[i:i + 1, :])
    bidx = pl.program_id(0)
    for s in range(NS):
        sig_ref[pl.ds(bidx * NS + s, 1), :] = \
            jnp.mean(x[s * NPAD:s * NPAD + NPS], axis=0, keepdims=True)

    @pl.when(bidx == B // NS - 1)
    def _():
        hc = jnp.dot(sig_ref[...], w1_ref[...],
                     preferred_element_type=jnp.float32) + b1_ref[...]
        hc = _ln_relu(hc, cg_ref[...], cbe_ref[...])
        out_ref[...] = jnp.dot(hc, w2_ref[...],
                               preferred_element_type=jnp.float32) + b2_ref[...]


def kernel(iq_signal, params):
    iqg = iq_signal.reshape(B // NS, NS, 2, L)

    bde = jnp.asarray(_BDE_NP).astype(jnp.bfloat16)
    bd0 = jnp.asarray(_BD0_NP).astype(jnp.bfloat16)
    sw = jnp.asarray(_SW_NP).astype(jnp.bfloat16)
    bds = jnp.asarray(_BDS_NP).astype(jnp.bfloat16)
    bds0 = jnp.asarray(_BDS0_NP).astype(jnp.bfloat16)
    w0 = jnp.concatenate([params['sage_Wl_0'], params['sage_Wr_0']],
                         axis=1).astype(jnp.bfloat16)
    wl = jnp.stack([params['sage_Wl_%d' % i]
                    for i in range(1, NL)]).astype(jnp.bfloat16)
    wr = jnp.stack([params['sage_Wr_%d' % i]
                    for i in range(1, NL)]).astype(jnp.bfloat16)
    b_all = jnp.stack([params['sage_b_%d' % i] for i in range(NL)])
    g_all = jnp.stack([params['ln_g_%d' % i] for i in range(NL)])
    bb_all = jnp.stack([params['ln_b_%d' % i] for i in range(NL)])

    logits = pl.pallas_call(
        _gnn_kernel,
        grid=(B // NS,),
        in_specs=[
            pl.BlockSpec((1, NS, 2, L), lambda b: (b, 0, 0, 0)),
            pl.BlockSpec((2, 2 * H), lambda b: (0, 0)),
            pl.BlockSpec((BT, YW), lambda b: (0, 0)),
            pl.BlockSpec((BT, YW), lambda b: (0, 0)),
            pl.BlockSpec((BT, YW), lambda b: (0, 0)),
            pl.BlockSpec((BT, 2 * BT), lambda b: (0, 0)),
            pl.BlockSpec((BT, BT), lambda b: (0, 0)),
            pl.BlockSpec((NL - 1, H, H), lambda b: (0, 0, 0)),
            pl.BlockSpec((NL - 1, H, H), lambda b: (0, 0, 0)),
            pl.BlockSpec((NL, H), lambda b: (0, 0)),
            pl.BlockSpec((NL, H), lambda b: (0, 0)),
            pl.BlockSpec((NL, H), lambda b: (0, 0)),
            pl.BlockSpec((H, H), lambda b: (0, 0)),
            pl.BlockSpec((1, H), lambda b: (0, 0)),
            pl.BlockSpec((1, H), lambda b: (0, 0)),
            pl.BlockSpec((1, H), lambda b: (0, 0)),
            pl.BlockSpec((H, NC), lambda b: (0, 0)),
            pl.BlockSpec((1, NC), lambda b: (0, 0)),
        ],
        out_specs=pl.BlockSpec((B, NC), lambda b: (0, 0)),
        out_shape=jax.ShapeDtypeStruct((B, NC), jnp.float32),
        scratch_shapes=[pltpu.VMEM((B, H), jnp.float32)],
    )(iqg, w0, sw, bds, bds0, bde, bd0, wl, wr, b_all, g_all, bb_all,
      params['cls_W1'], params['cls_b1'][None, :],
      params['cls_g'][None, :], params['cls_be'][None, :],
      params['cls_W2'], params['cls_b2'][None, :])
    return logits


# speedup vs baseline: 1.3659x; 1.0233x over previous
import numpy as np
import jax
import jax.numpy as jnp
from jax.experimental import pallas as pl
from jax.experimental.pallas import tpu as pltpu

B = 16
L = 1024
PL = 16
PS = 4
LW = 4
H = 128
NC = 8
NL = 4
P = (L - PL) // PS + 1
NPS = P * PL
NPAD = 4096
BT = 256
NT = NPAD // BT
NS = 4
NR = NS * NPAD


def _build_agg_mats():
    bd = np.zeros((BT, BT), dtype=np.float64)
    for r in range(BT):
        j = r % PL
        p0 = r - j
        for i in range(PL):
            if 0 < abs(i - j) <= LW:
                bd[r, p0 + i] = 1.0
        if j == 0 and r > 0:
            bd[r, r - 1] = 1.0
    deg = np.minimum(np.arange(BT) % PL, LW) \
        + np.minimum(PL - 1 - np.arange(BT) % PL, LW) \
        + ((np.arange(BT) % PL) == 0)
    bds = bd / deg[:, None]
    bd0 = bds.copy()
    bd0[0, :] = bd[0, :] / (deg[0] - 1)
    return bd0.astype(np.float32), bds.astype(np.float32)

_BD0_NP, _BDSQ_NP = _build_agg_mats()

_CMS_NP = np.where((np.arange(NS * NPAD) % BT == 0)
                   & (np.arange(NS * NPAD) % NPAD != 0),
                   1.0 / 5.0, 0.0)[:, None].astype(np.float32)

YW = 128
YOFF = 32


def _build_window_mats():
    sw = np.zeros((BT, YW), dtype=np.float64)
    bds = np.zeros((BT, YW), dtype=np.float64)
    bds0 = np.zeros((BT, YW), dtype=np.float64)

    def ycol(r):
        return 4 * (r // PL) + (r % PL) + YOFF

    deg = np.minimum(np.arange(BT) % PL, LW) \
        + np.minimum(PL - 1 - np.arange(BT) % PL, LW) \
        + ((np.arange(BT) % PL) == 0)
    for r in range(BT):
        sw[r, ycol(r)] = 1.0
        j = r % PL
        p0 = r - j
        for i in range(PL):
            if 0 < abs(i - j) <= LW:
                bds[r, ycol(p0 + i)] += 1.0 / deg[r]
                bds0[r, ycol(p0 + i)] += 1.0 / (deg[r] if r > 0 else deg[r] - 1)
        if j == 0:
            bds[r, ycol(r - 1)] += 1.0 / deg[r]
            if r > 0:
                bds0[r, ycol(r - 1)] += 1.0 / deg[r]
    return (sw.astype(np.float32), bds.astype(np.float32),
            bds0.astype(np.float32))

_SW_NP, _BDS_NP, _BDS0_NP = _build_window_mats()


def _ln_relu(h, g, b):
    mu = jnp.mean(h, axis=-1, keepdims=True)
    var = jnp.mean((h - mu) ** 2, axis=-1, keepdims=True)
    y = (h - mu) * jax.lax.rsqrt(var + 1e-5) * g + b
    return jnp.maximum(y, 0.0)


def _banded_mean_bf16(ub, u_f32, bds_ref, bd0_ref, cms):
    parts = []
    for t in range(NS * NT):
        bd = bd0_ref[...] if t % NT == 0 else bds_ref[...]
        parts.append(jnp.dot(bd, ub[t * BT:(t + 1) * BT],
                             preferred_element_type=jnp.float32))
    agg = jnp.concatenate(parts, axis=0)
    shift = jnp.concatenate([u_f32[NR - 1:], u_f32[:NR - 1]], axis=0)
    return agg + cms * shift


def _gnn_kernel(iq_ref, w0_ref, sw_ref, bds_ref, bds0_ref,
                bdsq_ref, bd0_ref, cms_ref, wl_ref, wr_ref,
                b_ref, g_ref, bb_ref,
                w1_ref, b1_ref, cg_ref, cbe_ref, w2_ref, b2_ref,
                out_ref, sig_ref):
    mean_parts = []
    v_parts = []
    zpad = jnp.zeros((YOFF, 2 * H), jnp.float32)
    for s in range(NS):
        yuv = jax.lax.dot_general(
            iq_ref[0, s].astype(jnp.bfloat16), w0_ref[...],
            (((0,), (0,)), ((), ())),
            preferred_element_type=jnp.float32)
        yb = jnp.concatenate([zpad, yuv, zpad], axis=0).astype(jnp.bfloat16)
        for t in range(NT):
            win = yb[64 * t:64 * t + YW]
            bdsm = bds0_ref[...] if t == 0 else bds_ref[...]
            mean_parts.append(jnp.dot(bdsm, win[:, :H],
                                      preferred_element_type=jnp.float32))
            v_parts.append(jnp.dot(sw_ref[...], win[:, H:],
                                   preferred_element_type=jnp.float32))
    h = jnp.concatenate(mean_parts, axis=0) \
        + jnp.concatenate(v_parts, axis=0) + b_ref[0:1, :]
    x = _ln_relu(h, g_ref[0:1, :], bb_ref[0:1, :])
    cms = cms_ref[...]
    for i in range(1, NL):
        xb = x.astype(jnp.bfloat16)
        m = _banded_mean_bf16(xb, x, bdsq_ref, bd0_ref,
                              cms).astype(jnp.bfloat16)
        h = jnp.dot(m, wl_ref[i - 1], preferred_element_type=jnp.float32) \
            + jnp.dot(xb, wr_ref[i - 1], preferred_element_type=jnp.float32) \
            + b_ref[i:i + 1, :]
        x = x + _ln_relu(h, g_ref[i:i + 1, :], bb_ref[i:i + 1, :])
    bidx = pl.program_id(0)
    for s in range(NS):
        sig_ref[pl.ds(bidx * NS + s, 1), :] = \
            jnp.mean(x[s * NPAD:s * NPAD + NPS], axis=0, keepdims=True)

    @pl.when(bidx == B // NS - 1)
    def _():
        hc = jnp.dot(sig_ref[...], w1_ref[...],
                     preferred_element_type=jnp.float32) + b1_ref[...]
        hc = _ln_relu(hc, cg_ref[...], cbe_ref[...])
        out_ref[...] = jnp.dot(hc, w2_ref[...],
                               preferred_element_type=jnp.float32) + b2_ref[...]


def kernel(iq_signal, params):
    iqg = iq_signal.reshape(B // NS, NS, 2, L)

    bdsq = jnp.asarray(_BDSQ_NP).astype(jnp.bfloat16)
    bd0 = jnp.asarray(_BD0_NP).astype(jnp.bfloat16)
    cmsv = jnp.asarray(_CMS_NP)
    sw = jnp.asarray(_SW_NP).astype(jnp.bfloat16)
    bds = jnp.asarray(_BDS_NP).astype(jnp.bfloat16)
    bds0 = jnp.asarray(_BDS0_NP).astype(jnp.bfloat16)
    w0 = jnp.concatenate([params['sage_Wl_0'], params['sage_Wr_0']],
                         axis=1).astype(jnp.bfloat16)
    wl = jnp.stack([params['sage_Wl_%d' % i]
                    for i in range(1, NL)]).astype(jnp.bfloat16)
    wr = jnp.stack([params['sage_Wr_%d' % i]
                    for i in range(1, NL)]).astype(jnp.bfloat16)
    b_all = jnp.stack([params['sage_b_%d' % i] for i in range(NL)])
    g_all = jnp.stack([params['ln_g_%d' % i] for i in range(NL)])
    bb_all = jnp.stack([params['ln_b_%d' % i] for i in range(NL)])

    logits = pl.pallas_call(
        _gnn_kernel,
        grid=(B // NS,),
        in_specs=[
            pl.BlockSpec((1, NS, 2, L), lambda b: (b, 0, 0, 0)),
            pl.BlockSpec((2, 2 * H), lambda b: (0, 0)),
            pl.BlockSpec((BT, YW), lambda b: (0, 0)),
            pl.BlockSpec((BT, YW), lambda b: (0, 0)),
            pl.BlockSpec((BT, YW), lambda b: (0, 0)),
            pl.BlockSpec((BT, BT), lambda b: (0, 0)),
            pl.BlockSpec((BT, BT), lambda b: (0, 0)),
            pl.BlockSpec((NR, 1), lambda b: (0, 0)),
            pl.BlockSpec((NL - 1, H, H), lambda b: (0, 0, 0)),
            pl.BlockSpec((NL - 1, H, H), lambda b: (0, 0, 0)),
            pl.BlockSpec((NL, H), lambda b: (0, 0)),
            pl.BlockSpec((NL, H), lambda b: (0, 0)),
            pl.BlockSpec((NL, H), lambda b: (0, 0)),
            pl.BlockSpec((H, H), lambda b: (0, 0)),
            pl.BlockSpec((1, H), lambda b: (0, 0)),
            pl.BlockSpec((1, H), lambda b: (0, 0)),
            pl.BlockSpec((1, H), lambda b: (0, 0)),
            pl.BlockSpec((H, NC), lambda b: (0, 0)),
            pl.BlockSpec((1, NC), lambda b: (0, 0)),
        ],
        out_specs=pl.BlockSpec((B, NC), lambda b: (0, 0)),
        out_shape=jax.ShapeDtypeStruct((B, NC), jnp.float32),
        scratch_shapes=[pltpu.VMEM((B, H), jnp.float32)],
    )(iqg, w0, sw, bds, bds0, bdsq, bd0, cmsv, wl, wr, b_all, g_all, bb_all,
      params['cls_W1'], params['cls_b1'][None, :],
      params['cls_g'][None, :], params['cls_be'][None, :],
      params['cls_W2'], params['cls_b2'][None, :])
    return logits
